# baseline (speedup 1.0000x reference)
import functools

import jax
import jax.numpy as jnp
from jax import lax
from jax.experimental import pallas as pl
from jax.experimental.pallas import tpu as pltpu
from jax.experimental.pallas import tpu_sc as plsc

_T = 4096
_D = 2048
_R = 512
_FF = 768
_E = 8
_K = 2
_B = 256
_P = _T * _K + _E * _B
_NB = _P // _B
_NW = 32
_GCH = 64
_GNB = 5
_CCH = 8
_CNB = 3
_CNC = (_T // _NW) // _CCH


def _lowrank_proj_kernel(x_ref, vh_ref, out_ref):
    low = lax.dot_general(
        x_ref[...].astype(jnp.bfloat16), vh_ref[...], (((1,), (1,)), ((), ())),
        preferred_element_type=jnp.float32)
    b0 = lax.bitcast_convert_type(
        low[:, :_R].astype(jnp.bfloat16), jnp.uint16).astype(jnp.int32)
    b1 = lax.bitcast_convert_type(
        low[:, _R:].astype(jnp.bfloat16), jnp.uint16).astype(jnp.int32)
    out_ref[...] = b0 | (b1 << 16)


def _expert_block_kernel(be_ref, glow_ref, u_ref, d_ref, w_ref, y_ref):
    b = pl.program_id(0)

    @pl.when(b < be_ref[_NB])
    def _():
        w = glow_ref[...]
        g1 = be_ref[b] >= (_E // 2)
        fbits = jnp.where(g1, w & jnp.int32(-65536), w << 16)
        glow = lax.bitcast_convert_type(fbits, jnp.float32).astype(jnp.bfloat16)
        gu = lax.dot_general(glow, u_ref[0],
                             (((1,), (1,)), ((), ())),
                             preferred_element_type=jnp.float32)
        gate = gu[:, :_FF]
        up = gu[:, _FF:]
        h = gate * jax.nn.sigmoid(gate) * up
        y = lax.dot_general(h.astype(jnp.bfloat16), d_ref[0],
                            (((1,), (1,)), ((), ())),
                            preferred_element_type=jnp.float32)
        y_ref[...] = y * w_ref[0, 0][:, None]


def _sc_gather(lowp, gidx):
    mesh = plsc.VectorSubcoreMesh(core_axis_name="c", subcore_axis_name="s")
    rows_per_t = _P // 16
    nch = rows_per_t // _GCH
    hc = _R // 4
    stage_rows = _T // 16

    @functools.partial(
        pl.kernel, mesh=mesh,
        out_type=jax.ShapeDtypeStruct((_P, _R), jnp.int32),
        scratch_types=(
            [pltpu.VMEM_SHARED((_T, hc), jnp.int32)]
            + [pltpu.VMEM((rows_per_t,), jnp.int32)]
            + [pltpu.VMEM((_GCH, hc), jnp.int32) for _ in range(_GNB)]
            + [pltpu.SemaphoreType.DMA for _ in range(2 * _GNB)]
        ),
    )
    def k(lowp_hbm, gidx_hbm, glow_hbm, *refs):
        tab = refs[0]
        idx = refs[1]
        bufs = refs[2:2 + _GNB]
        gs = refs[2 + _GNB:2 + 2 * _GNB]
        ws = refs[2 + 2 * _GNB:2 + 3 * _GNB]
        cid = lax.axis_index("c")
        sid = lax.axis_index("s")
        base = sid * rows_per_t
        pltpu.sync_copy(gidx_hbm.at[pl.ds(base, rows_per_t)], idx)
        for half in range(2):
            col = (2 * half) * hc + cid * hc
            pltpu.sync_copy(
                lowp_hbm.at[pl.ds(sid * stage_rows, stage_rows),
                            pl.ds(col, hc)],
                tab.at[pl.ds(sid * stage_rows, stage_rows)])
            plsc.subcore_barrier()

            def fire(c):
                b = c % _GNB
                return pltpu.async_copy(
                    tab.at[idx.at[pl.ds(c * _GCH, _GCH)]], bufs[b], gs[b])

            gcp = [None] * nch
            wcp = [None] * nch
            for c in range(min(_GNB, nch)):
                gcp[c] = fire(c)
            for c in range(nch):
                b = c % _GNB
                gcp[c].wait()
                wcp[c] = pltpu.async_copy(
                    bufs[b],
                    glow_hbm.at[pl.ds(base + c * _GCH, _GCH), pl.ds(col, hc)],
                    ws[b])
                if c + _GNB < nch:
                    wcp[c].wait()
                    gcp[c + _GNB] = fire(c + _GNB)
            for c in range(max(0, nch - _GNB), nch):
                wcp[c].wait()
            plsc.subcore_barrier()

    return k(lowp, gidx)


def _sc_combine(y, pos0, pos1):
    mesh = plsc.VectorSubcoreMesh(core_axis_name="c", subcore_axis_name="s")
    tok_per_w = _T // _NW

    @functools.partial(
        pl.kernel, mesh=mesh,
        out_type=jax.ShapeDtypeStruct((_T, _D), jnp.float32),
        scratch_types=(
            [pltpu.VMEM((tok_per_w,), jnp.int32) for _ in range(2)]
            + [pltpu.VMEM((_CCH, _D), jnp.float32) for _ in range(2 * _CNB)]
            + [pltpu.SemaphoreType.DMA for _ in range(3 * _CNB)]
        ),
    )
    def k(y_hbm, p0_hbm, p1_hbm, out_hbm, *refs):
        i0, i1 = refs[0], refs[1]
        av = refs[2:2 + _CNB]
        bv = refs[2 + _CNB:2 + 2 * _CNB]
        gsa = refs[2 + 2 * _CNB:2 + 3 * _CNB]
        gsb = refs[2 + 3 * _CNB:2 + 4 * _CNB]
        wsm = refs[2 + 4 * _CNB:2 + 5 * _CNB]
        wid = lax.axis_index("s") * 2 + lax.axis_index("c")
        base = wid * tok_per_w
        pltpu.sync_copy(p0_hbm.at[pl.ds(base, tok_per_w)], i0)
        pltpu.sync_copy(p1_hbm.at[pl.ds(base, tok_per_w)], i1)

        def fire(c):
            s = c % _CNB
            o = pl.ds(c * _CCH, _CCH)
            ga = pltpu.async_copy(y_hbm.at[i0.at[o]], av[s], gsa[s])
            gb = pltpu.async_copy(y_hbm.at[i1.at[o]], bv[s], gsb[s])
            return ga, gb

        gcp = [None] * _CNC
        wcp = [None] * _CNC
        for c in range(min(_CNB, _CNC)):
            gcp[c] = fire(c)
        for c in range(_CNC):
            s = c % _CNB
            gcp[c][0].wait()
            gcp[c][1].wait()
            for r in range(_CCH):
                def body(j, _, r=r, s=s):
                    sl = pl.ds(j * 16, 16)
                    av[s][r, sl] = av[s][r, sl] + bv[s][r, sl]
                    return 0
                lax.fori_loop(0, _D // 16, body, 0, unroll=8)
            wcp[c] = pltpu.async_copy(
                av[s], out_hbm.at[pl.ds(base + c * _CCH, _CCH)], wsm[s])
            if c + _CNB < _CNC:
                wcp[c].wait()
                gcp[c + _CNB] = fire(c + _CNB)
        for c in range(max(0, _CNC - _CNB), _CNC):
            wcp[c].wait()

    return k(y, pos0, pos1)


def _routing(idx, w):
    tk = _T * _K
    flat_e = idx.reshape(-1)
    flat_w = w.reshape(-1)
    tok_of_pair = jnp.arange(tk, dtype=jnp.int32) // _K
    order = jnp.argsort(flat_e)
    inv = jnp.argsort(order).astype(jnp.int32)
    counts = jnp.bincount(flat_e, length=_E).astype(jnp.int32)
    off = jnp.concatenate(
        [jnp.zeros((1,), jnp.int32), jnp.cumsum(counts)[:-1].astype(jnp.int32)])
    padc = ((counts + _B - 1) // _B) * _B
    pad_off = jnp.concatenate(
        [jnp.zeros((1,), jnp.int32), jnp.cumsum(padc)[:-1].astype(jnp.int32)])
    starts = jnp.arange(_NB, dtype=jnp.int32) * _B
    be = (jnp.searchsorted(pad_off, starts, side='right') - 1).astype(jnp.int32)
    nact = (pad_off[_E - 1] + padc[_E - 1]) // _B
    be_ext = jnp.concatenate([be, nact.astype(jnp.int32)[None]])
    slot_e = jnp.repeat(be, _B)
    slot_i = jnp.arange(_P, dtype=jnp.int32)
    r_un = slot_i - pad_off[slot_e] + off[slot_e]
    valid = (slot_i - pad_off[slot_e]) < counts[slot_e]
    pair = order[jnp.clip(r_un, 0, tk - 1)]
    slot_tok = jnp.where(valid, tok_of_pair[pair], 0)
    slot_w = jnp.where(valid, flat_w[pair], 0.0)
    gidx = slot_tok
    pos = (pad_off[flat_e] + (inv - off[flat_e])).reshape(_T, _K)
    return be_ext, slot_w, gidx, pos


def kernel(hidden_states, top_k_weights, vh0, vh1, u0, u1, d0, d1, top_k_index):
    idx = top_k_index.astype(jnp.int32)
    be, slot_w, gidx, pos = _routing(idx, top_k_weights)

    vhcat = jnp.concatenate([vh0, vh1], axis=0).astype(jnp.bfloat16)
    tb = 512
    low = pl.pallas_call(
        _lowrank_proj_kernel,
        grid=(_T // tb,),
        in_specs=[pl.BlockSpec((tb, _D), lambda i: (i, 0)),
                  pl.BlockSpec((2 * _R, _D), lambda i: (0, 0))],
        out_specs=pl.BlockSpec((tb, _R), lambda i: (i, 0)),
        out_shape=jax.ShapeDtypeStruct((_T, _R), jnp.int32),
    )(hidden_states, vhcat)

    glow = _sc_gather(low, gidx)

    u_all = jnp.concatenate([u0, u1], axis=0).astype(jnp.bfloat16)
    d_all = jnp.concatenate([d0, d1], axis=0).astype(jnp.bfloat16)
    w3 = slot_w.reshape(_NB, 1, _B)
    grid_spec = pltpu.PrefetchScalarGridSpec(
        num_scalar_prefetch=1,
        grid=(_NB,),
        in_specs=[
            pl.BlockSpec((_B, _R), lambda b, be_ref: (b, 0)),
            pl.BlockSpec((1, 2 * _FF, _R), lambda b, be_ref: (be_ref[b], 0, 0)),
            pl.BlockSpec((1, _D, _FF), lambda b, be_ref: (be_ref[b], 0, 0)),
            pl.BlockSpec((1, 1, _B), lambda b, be_ref: (b, 0, 0)),
        ],
        out_specs=pl.BlockSpec((_B, _D), lambda b, be_ref: (b, 0)),
    )
    y = pl.pallas_call(
        _expert_block_kernel,
        grid_spec=grid_spec,
        out_shape=jax.ShapeDtypeStruct((_P, _D), jnp.float32),
    )(be, glow, u_all, d_all, w3)

    return _sc_combine(y, pos[:, 0], pos[:, 1])


# speedup vs baseline: 1.2892x; 1.2892x over previous
import functools

import jax
import jax.numpy as jnp
from jax import lax
from jax.experimental import pallas as pl
from jax.experimental.pallas import tpu as pltpu
from jax.experimental.pallas import tpu_sc as plsc

_T = 4096
_D = 2048
_R = 512
_FF = 768
_E = 8
_K = 2
_B = 256
_P = _T * _K + _E * _B
_NB = _P // _B
_NW = 32
_GCH = 64
_GNB = 5
_CCH = 8
_CNB = 3
_CNC = (_T // _NW) // _CCH


def _lowrank_proj_kernel(x_ref, vh_ref, out_ref):
    low = lax.dot_general(
        x_ref[...].astype(jnp.bfloat16), vh_ref[...], (((1,), (1,)), ((), ())),
        preferred_element_type=jnp.float32)
    b0 = lax.bitcast_convert_type(
        low[:, :_R].astype(jnp.bfloat16), jnp.uint16).astype(jnp.int32)
    b1 = lax.bitcast_convert_type(
        low[:, _R:].astype(jnp.bfloat16), jnp.uint16).astype(jnp.int32)
    out_ref[...] = b0 | (b1 << 16)


def _expert_block_kernel(be_ref, glow_ref, u_ref, d_ref, w_ref, y_ref):
    b = pl.program_id(0)

    @pl.when(b < be_ref[_NB])
    def _():
        w = glow_ref[...]
        g1 = be_ref[b] >= (_E // 2)
        fbits = jnp.where(g1, w & jnp.int32(-65536), w << 16)
        glow = lax.bitcast_convert_type(fbits, jnp.float32).astype(jnp.bfloat16)
        gu = lax.dot_general(glow, u_ref[0],
                             (((1,), (1,)), ((), ())),
                             preferred_element_type=jnp.float32)
        gate = gu[:, :_FF]
        up = gu[:, _FF:]
        h = gate * jax.nn.sigmoid(gate) * up
        y = lax.dot_general(h.astype(jnp.bfloat16), d_ref[0],
                            (((1,), (1,)), ((), ())),
                            preferred_element_type=jnp.float32)
        y_ref[...] = y * w_ref[0, 0][:, None]


def _sc_gather(lowp, gidx):
    mesh = plsc.VectorSubcoreMesh(core_axis_name="c", subcore_axis_name="s")
    rows_per_t = _P // 16
    nch = rows_per_t // _GCH
    hc = _R // 4
    stage_rows = _T // 16

    @functools.partial(
        pl.kernel, mesh=mesh,
        out_type=jax.ShapeDtypeStruct((_P, _R), jnp.int32),
        scratch_types=(
            [pltpu.VMEM_SHARED((_T, hc), jnp.int32)]
            + [pltpu.VMEM((rows_per_t,), jnp.int32)]
            + [pltpu.VMEM((_GCH, hc), jnp.int32) for _ in range(_GNB)]
            + [pltpu.SemaphoreType.DMA for _ in range(2 * _GNB)]
        ),
    )
    def k(lowp_hbm, gidx_hbm, glow_hbm, *refs):
        tab = refs[0]
        idx = refs[1]
        bufs = refs[2:2 + _GNB]
        gs = refs[2 + _GNB:2 + 2 * _GNB]
        ws = refs[2 + 2 * _GNB:2 + 3 * _GNB]
        cid = lax.axis_index("c")
        sid = lax.axis_index("s")
        base = sid * rows_per_t
        pltpu.sync_copy(gidx_hbm.at[pl.ds(base, rows_per_t)], idx)
        for half in range(2):
            col = (2 * half) * hc + cid * hc
            pltpu.sync_copy(
                lowp_hbm.at[pl.ds(sid * stage_rows, stage_rows),
                            pl.ds(col, hc)],
                tab.at[pl.ds(sid * stage_rows, stage_rows)])
            plsc.subcore_barrier()

            def fire(c):
                b = c % _GNB
                return pltpu.async_copy(
                    tab.at[idx.at[pl.ds(c * _GCH, _GCH)]], bufs[b], gs[b])

            gcp = [None] * nch
            wcp = [None] * nch
            for c in range(min(_GNB, nch)):
                gcp[c] = fire(c)
            for c in range(nch):
                b = c % _GNB
                gcp[c].wait()
                wcp[c] = pltpu.async_copy(
                    bufs[b],
                    glow_hbm.at[pl.ds(base + c * _GCH, _GCH), pl.ds(col, hc)],
                    ws[b])
                if c + _GNB < nch:
                    wcp[c].wait()
                    gcp[c + _GNB] = fire(c + _GNB)
            for c in range(max(0, nch - _GNB), nch):
                wcp[c].wait()
            plsc.subcore_barrier()

    return k(lowp, gidx)


def _sc_combine(y, pos0, pos1):
    mesh = plsc.VectorSubcoreMesh(core_axis_name="c", subcore_axis_name="s")
    tok_per_w = _T // _NW

    @functools.partial(
        pl.kernel, mesh=mesh,
        out_type=jax.ShapeDtypeStruct((_T, _D), jnp.float32),
        scratch_types=(
            [pltpu.VMEM((tok_per_w,), jnp.int32) for _ in range(2)]
            + [pltpu.VMEM((_CCH, _D), jnp.float32) for _ in range(2 * _CNB)]
            + [pltpu.SemaphoreType.DMA for _ in range(3 * _CNB)]
        ),
    )
    def k(y_hbm, p0_hbm, p1_hbm, out_hbm, *refs):
        i0, i1 = refs[0], refs[1]
        av = refs[2:2 + _CNB]
        bv = refs[2 + _CNB:2 + 2 * _CNB]
        gsa = refs[2 + 2 * _CNB:2 + 3 * _CNB]
        gsb = refs[2 + 3 * _CNB:2 + 4 * _CNB]
        wsm = refs[2 + 4 * _CNB:2 + 5 * _CNB]
        wid = lax.axis_index("s") * 2 + lax.axis_index("c")
        base = wid * tok_per_w
        pltpu.sync_copy(p0_hbm.at[pl.ds(base, tok_per_w)], i0)
        pltpu.sync_copy(p1_hbm.at[pl.ds(base, tok_per_w)], i1)

        def fire(c):
            s = c % _CNB
            o = pl.ds(c * _CCH, _CCH)
            ga = pltpu.async_copy(y_hbm.at[i0.at[o]], av[s], gsa[s])
            gb = pltpu.async_copy(y_hbm.at[i1.at[o]], bv[s], gsb[s])
            return ga, gb

        gcp = [None] * _CNC
        wcp = [None] * _CNC
        for c in range(min(_CNB, _CNC)):
            gcp[c] = fire(c)
        for c in range(_CNC):
            s = c % _CNB
            gcp[c][0].wait()
            gcp[c][1].wait()
            for r in range(_CCH):
                def body(j, _, r=r, s=s):
                    sl = pl.ds(j * 16, 16)
                    av[s][r, sl] = av[s][r, sl] + bv[s][r, sl]
                    return 0
                lax.fori_loop(0, _D // 16, body, 0, unroll=8)
            wcp[c] = pltpu.async_copy(
                av[s], out_hbm.at[pl.ds(base + c * _CCH, _CCH)], wsm[s])
            if c + _CNB < _CNC:
                wcp[c].wait()
                gcp[c + _CNB] = fire(c + _CNB)
        for c in range(max(0, _CNC - _CNB), _CNC):
            wcp[c].wait()

    return k(y, pos0, pos1)


def _sc_routing(idxf, wf, tokf):
    mesh = plsc.VectorSubcoreMesh(core_axis_name="c", subcore_axis_name="s")
    tk = _T * _K
    cpt = tk // 16
    nv = cpt // 16
    stripe = _P // 16

    @functools.partial(
        pl.kernel, mesh=mesh,
        out_type=(
            jax.ShapeDtypeStruct((48,), jnp.int32),
            jax.ShapeDtypeStruct((_P,), jnp.int32),
            jax.ShapeDtypeStruct((_P,), jnp.float32),
            jax.ShapeDtypeStruct((tk // 128, 128), jnp.int32),
        ),
        scratch_types=[
            pltpu.VMEM_SHARED((_P,), jnp.int32),
            pltpu.VMEM_SHARED((_P,), jnp.float32),
            pltpu.VMEM_SHARED((256,), jnp.int32),
            pltpu.VMEM((cpt,), jnp.int32),
            pltpu.VMEM((cpt,), jnp.float32),
            pltpu.VMEM((cpt,), jnp.int32),
            pltpu.VMEM((4, 128), jnp.int32),
            pltpu.VMEM((16,), jnp.int32),
            pltpu.VMEM((16,), jnp.int32),
            pltpu.VMEM((256,), jnp.int32),
            pltpu.VMEM((stripe,), jnp.int32),
            pltpu.VMEM((stripe,), jnp.float32),
            pltpu.VMEM((48,), jnp.int32),
        ],
        compiler_params=pltpu.CompilerParams(needs_layout_passes=False),
    )
    def k(idxf_hbm, wf_hbm, tokf_hbm, be_hbm, gidx_hbm, sw_hbm, pos_hbm,
          tok_s, w_s, cnt_s, idx_v, w_v, tok_v, pos_v, hist, pref_r, cnt_all,
          z32, zf32, be_v):
        cid = lax.axis_index("c")
        sid = lax.axis_index("s")
        iota = lax.iota(jnp.int32, 16)
        zero16 = jnp.zeros((16,), jnp.int32)
        one16 = jnp.ones((16,), jnp.int32)

        def lane_bcast(vec, lane):
            return vec.at[jnp.full((16,), lane, jnp.int32)].get(
                mode="promise_in_bounds")

        pltpu.sync_copy(idxf_hbm.at[pl.ds(sid * cpt, cpt)], idx_v)
        pltpu.sync_copy(wf_hbm.at[pl.ds(sid * cpt, cpt)], w_v)
        pltpu.sync_copy(tokf_hbm.at[pl.ds(sid * cpt, cpt)], tok_v)

        for j in range(stripe // 16):
            z32[pl.ds(j * 16, 16)] = zero16
            zf32[pl.ds(j * 16, 16)] = jnp.zeros((16,), jnp.float32)
        pltpu.sync_copy(z32, tok_s.at[pl.ds(sid * stripe, stripe)])
        pltpu.sync_copy(zf32, w_s.at[pl.ds(sid * stripe, stripe)])

        evs = [jnp.full((16,), e, jnp.int32) for e in range(_E)]
        macc = [zero16] * _E
        for j in range(nv):
            v = idx_v[pl.ds(j * 16, 16)]
            for e in range(_E):
                macc[e] = macc[e] + jnp.where(v == evs[e], one16, zero16)
        cnt = zero16
        for e in range(_E):
            bc = lane_bcast(lax.cumsum(macc[e], axis=0), 15)
            cnt = cnt + jnp.where(iota == evs[e], bc, zero16)
        hist[...] = cnt
        pltpu.sync_copy(hist, cnt_s.at[pl.ds(sid * 16, 16)])
        plsc.subcore_barrier()

        pltpu.sync_copy(cnt_s, cnt_all)
        pref_r[...] = zero16
        tot = jnp.zeros((16,), jnp.int32)
        for s in range(16):
            tot = tot + cnt_all[pl.ds(s * 16, 16)]

            @pl.when(s < sid)
            def _(s=s):
                pref_r[...] = pref_r[...] + cnt_all[pl.ds(s * 16, 16)]
        bm1 = jnp.full((16,), _B - 1, jnp.int32)
        bmask = jnp.full((16,), -_B, jnp.int32)
        pad_l = (tot + bm1) & bmask
        pad_off = lax.cumsum(pad_l, axis=0) - pad_l
        my_start = pad_off + pref_r[...]

        curv = my_start
        for j in range(nv):
            v = idx_v[pl.ds(j * 16, 16)]
            basev = curv.at[v].get(mode="promise_in_bounds")
            rankv = jnp.zeros((16,), jnp.int32)
            histv = zero16
            for e in range(_E):
                m = v == evs[e]
                cm = lax.cumsum(jnp.where(m, one16, zero16), axis=0)
                rankv = jnp.where(m, cm, rankv)
                histv = histv + jnp.where(iota == evs[e], lane_bcast(cm, 15),
                                          zero16)
            posv = basev + rankv - one16
            posv = jnp.minimum(jnp.maximum(posv, zero16),
                               jnp.full((16,), _P - 1, jnp.int32))
            curv = curv + histv
            pos_v[j // 8, pl.ds((j % 8) * 16, 16)] = posv

        for i in range(4):
            pltpu.sync_copy(tok_v.at[pl.ds(i * 128, 128)],
                            tok_s.at[pos_v.at[i]])
            pltpu.sync_copy(w_v.at[pl.ds(i * 128, 128)],
                            w_s.at[pos_v.at[i]])
        plsc.subcore_barrier()

        @pl.when(sid == 0)
        def _():
            half = _P // 2
            pltpu.sync_copy(tok_s.at[pl.ds(cid * half, half)],
                            gidx_hbm.at[pl.ds(cid * half, half)])
            pltpu.sync_copy(w_s.at[pl.ds(cid * half, half)],
                            sw_hbm.at[pl.ds(cid * half, half)])

        @pl.when(cid == 0)
        def _():
            pltpu.sync_copy(pos_v, pos_hbm.at[pl.ds(sid * 4, 4)])

        @pl.when(jnp.logical_and(sid == 0, cid == 1))
        def _():
            endv = pad_off + pad_l
            sh8 = jnp.full((16,), 8, jnp.int32)
            nactv = lax.shift_right_logical(lane_bcast(endv, _E - 1), sh8)
            for r in range(3):
                bs = (iota + jnp.full((16,), r * 16, jnp.int32)) * jnp.full(
                    (16,), _B, jnp.int32)
                a = jnp.zeros((16,), jnp.int32)
                for e in range(_E):
                    pe = lane_bcast(pad_off, e)
                    a = a + jnp.where(pe <= bs, one16, zero16)
                bev = a - one16
                if r == 2:
                    bev = jnp.where(iota == jnp.full((16,), _NB - 32,
                                                     jnp.int32), nactv, bev)
                be_v[pl.ds(r * 16, 16)] = bev
            pltpu.sync_copy(be_v, be_hbm)

    return k(idxf, wf, tokf)


def kernel(hidden_states, top_k_weights, vh0, vh1, u0, u1, d0, d1, top_k_index):
    idxf = top_k_index.astype(jnp.int32).reshape(-1)
    wf = top_k_weights.reshape(-1)
    tokf = jnp.arange(_T * _K, dtype=jnp.int32) >> 1
    be, gidx, slot_w, pos2 = _sc_routing(idxf, wf, tokf)
    pos = pos2.reshape(_T, _K)

    vhcat = jnp.concatenate([vh0, vh1], axis=0).astype(jnp.bfloat16)
    tb = 512
    low = pl.pallas_call(
        _lowrank_proj_kernel,
        grid=(_T // tb,),
        in_specs=[pl.BlockSpec((tb, _D), lambda i: (i, 0)),
                  pl.BlockSpec((2 * _R, _D), lambda i: (0, 0))],
        out_specs=pl.BlockSpec((tb, _R), lambda i: (i, 0)),
        out_shape=jax.ShapeDtypeStruct((_T, _R), jnp.int32),
    )(hidden_states, vhcat)

    glow = _sc_gather(low, gidx)

    u_all = jnp.concatenate([u0, u1], axis=0).astype(jnp.bfloat16)
    d_all = jnp.concatenate([d0, d1], axis=0).astype(jnp.bfloat16)
    w3 = slot_w.reshape(_NB, 1, _B)
    grid_spec = pltpu.PrefetchScalarGridSpec(
        num_scalar_prefetch=1,
        grid=(_NB,),
        in_specs=[
            pl.BlockSpec((_B, _R), lambda b, be_ref: (b, 0)),
            pl.BlockSpec((1, 2 * _FF, _R), lambda b, be_ref: (be_ref[b], 0, 0)),
            pl.BlockSpec((1, _D, _FF), lambda b, be_ref: (be_ref[b], 0, 0)),
            pl.BlockSpec((1, 1, _B), lambda b, be_ref: (b, 0, 0)),
        ],
        out_specs=pl.BlockSpec((_B, _D), lambda b, be_ref: (b, 0)),
    )
    y = pl.pallas_call(
        _expert_block_kernel,
        grid_spec=grid_spec,
        out_shape=jax.ShapeDtypeStruct((_P, _D), jnp.float32),
    )(be, glow, u_all, d_all, w3)

    return _sc_combine(y, pos[:, 0], pos[:, 1])
